# Initial kernel scaffold; baseline (speedup 1.0000x reference)
import jax
import jax.numpy as jnp
from jax.experimental import pallas as pl


def kernel(x, sec_pos_label, in_sec_pos_label, pe):
    raise NotImplementedError("write your pallas kernel here")


# speedup vs baseline: 1.0022x; 1.0022x over previous
import functools

import jax
import jax.numpy as jnp
from jax import lax
from jax.experimental import pallas as pl
from jax.experimental.pallas import tpu as pltpu
from jax.experimental.pallas import tpu_sc as plsc

D_MODEL = 1024
LANES = 16
CHUNK = 32


@functools.cache
def _build_sc_call(n_rows: int, d: int):
    info = plsc.get_sparse_core_info()
    nw = info.num_cores * info.num_subcores
    rows_per_w = n_rows // nw
    n_chunks = rows_per_w // CHUNK
    assert rows_per_w % CHUNK == 0 and d % LANES == 0

    mesh = plsc.VectorSubcoreMesh(core_axis_name="c", subcore_axis_name="s")

    @functools.partial(
        pl.kernel,
        mesh=mesh,
        out_type=jax.ShapeDtypeStruct((n_rows, d), jnp.float32),
        scratch_types=[
            pltpu.VMEM((CHUNK,), jnp.int32),
            pltpu.VMEM((CHUNK,), jnp.int32),
            pltpu.VMEM((CHUNK, d), jnp.float32),
            pltpu.VMEM((CHUNK, d), jnp.float32),
            pltpu.VMEM((CHUNK, d), jnp.float32),
            pltpu.SemaphoreType.DMA,
        ],
    )
    def sc_kernel(x_hbm, ia_hbm, ib_hbm, pe_hbm, out_hbm,
                  ia_v, ib_v, x_v, ra_v, rb_v, sem):
        wid = lax.axis_index("s") * info.num_cores + lax.axis_index("c")
        base_w = wid * rows_per_w

        def chunk_body(ci, carry):
            base = base_w + ci * CHUNK
            pltpu.sync_copy(ia_hbm.at[pl.ds(base, CHUNK)], ia_v)
            pltpu.sync_copy(ib_hbm.at[pl.ds(base, CHUNK)], ib_v)
            cp_a = pltpu.async_copy(pe_hbm.at[ia_v], ra_v, sem)
            cp_b = pltpu.async_copy(pe_hbm.at[ib_v], rb_v, sem)
            pltpu.sync_copy(x_hbm.at[pl.ds(base, CHUNK)], x_v)
            cp_a.wait()
            cp_b.wait()

            def row_body(r, rcarry):
                for j in range(d // LANES):
                    s = j * LANES
                    x_v[r, pl.ds(s, LANES)] = (
                        x_v[r, pl.ds(s, LANES)]
                        + (ra_v[r, pl.ds(s, LANES)] + rb_v[r, pl.ds(s, LANES)])
                        * 0.001
                    )
                return rcarry

            lax.fori_loop(0, CHUNK, row_body, 0)
            pltpu.sync_copy(x_v, out_hbm.at[pl.ds(base, CHUNK)])
            return carry

        lax.fori_loop(0, n_chunks, chunk_body, 0)

    return sc_kernel


def kernel(x, sec_pos_label, in_sec_pos_label, pe):
    seq, batch, d = x.shape
    n_rows = seq * batch
    x2 = x.reshape(n_rows, d)
    ia = sec_pos_label.reshape(n_rows).astype(jnp.int32)
    ib = in_sec_pos_label.reshape(n_rows).astype(jnp.int32)
    pe2 = pe.reshape(pe.shape[0], d)
    out2 = _build_sc_call(n_rows, d)(x2, ia, ib, pe2)
    return out2.reshape(seq, batch, d)


# speedup vs baseline: 1.4102x; 1.4072x over previous
import functools

import jax
import jax.numpy as jnp
from jax import lax
from jax.experimental import pallas as pl
from jax.experimental.pallas import tpu as pltpu
from jax.experimental.pallas import tpu_sc as plsc

LANES = 16
CHUNK = 8
NBUF = 4


@functools.cache
def _build_sc_call(n_rows: int, d: int):
    info = plsc.get_sparse_core_info()
    nw = info.num_cores * info.num_subcores
    rows_per_w = n_rows // nw
    n_chunks = rows_per_w // CHUNK
    assert rows_per_w % CHUNK == 0 and d % LANES == 0
    assert n_chunks % NBUF == 0

    mesh = plsc.VectorSubcoreMesh(core_axis_name="c", subcore_axis_name="s")

    @functools.partial(
        pl.kernel,
        mesh=mesh,
        out_type=jax.ShapeDtypeStruct((n_rows, d), jnp.float32),
        scratch_types=[
            pltpu.VMEM((rows_per_w,), jnp.int32),
            pltpu.VMEM((rows_per_w,), jnp.int32),
            pltpu.VMEM((NBUF, CHUNK, d), jnp.float32),
            pltpu.VMEM((NBUF, CHUNK, d), jnp.float32),
            pltpu.VMEM((NBUF, CHUNK, d), jnp.float32),
        ]
        + [pltpu.SemaphoreType.DMA] * (2 * NBUF),
    )
    def sc_kernel(x_hbm, ia_hbm, ib_hbm, pe_hbm, out_hbm,
                  ia_v, ib_v, x_v, ra_v, rb_v, *sems):
        sem_in = sems[:NBUF]
        sem_out = sems[NBUF:]
        wid = lax.axis_index("s") * info.num_cores + lax.axis_index("c")
        base_w = wid * rows_per_w

        pltpu.sync_copy(ia_hbm.at[pl.ds(base_w, rows_per_w)], ia_v)
        pltpu.sync_copy(ib_hbm.at[pl.ds(base_w, rows_per_w)], ib_v)

        def issue_in(ci, b):
            off = ci * CHUNK
            pltpu.async_copy(
                pe_hbm.at[ia_v.at[pl.ds(off, CHUNK)]], ra_v.at[b], sem_in[b])
            pltpu.async_copy(
                pe_hbm.at[ib_v.at[pl.ds(off, CHUNK)]], rb_v.at[b], sem_in[b])
            pltpu.async_copy(
                x_hbm.at[pl.ds(base_w + off, CHUNK)], x_v.at[b], sem_in[b])

        issue_in(0, 0)

        def super_body(i, carry):
            for b in range(NBUF):
                ci = i * NBUF + b
                bn = (b + 1) % NBUF

                @pl.when(ci >= NBUF - 1)
                def _():
                    pltpu.make_async_copy(
                        x_v.at[bn], out_hbm.at[pl.ds(0, CHUNK)],
                        sem_out[bn]).wait()

                @pl.when(ci < n_chunks - 1)
                def _():
                    issue_in(ci + 1, bn)

                pltpu.make_async_copy(
                    x_hbm.at[pl.ds(0, CHUNK)], ra_v.at[b], sem_in[b]).wait()
                pltpu.make_async_copy(
                    x_hbm.at[pl.ds(0, CHUNK)], rb_v.at[b], sem_in[b]).wait()
                pltpu.make_async_copy(
                    x_hbm.at[pl.ds(0, CHUNK)], x_v.at[b], sem_in[b]).wait()

                def row_body(r, rcarry):
                    for j in range(d // LANES):
                        s = j * LANES
                        val = (ra_v[b, r, pl.ds(s, LANES)]
                               + rb_v[b, r, pl.ds(s, LANES)]) * 0.001
                        plsc.addupdate(x_v.at[b, r, pl.ds(s, LANES)], val)
                    return rcarry

                lax.fori_loop(0, CHUNK, row_body, 0)

                pltpu.async_copy(
                    x_v.at[b],
                    out_hbm.at[pl.ds(base_w + ci * CHUNK, CHUNK)],
                    sem_out[b])
            return carry

        lax.fori_loop(0, n_chunks // NBUF, super_body, 0)

        for ci in range(n_chunks - (NBUF - 1), n_chunks):
            b = ci % NBUF
            pltpu.make_async_copy(
                x_v.at[b], out_hbm.at[pl.ds(0, CHUNK)], sem_out[b]).wait()

    return sc_kernel


def kernel(x, sec_pos_label, in_sec_pos_label, pe):
    seq, batch, d = x.shape
    n_rows = seq * batch
    x2 = x.reshape(n_rows, d)
    ia = sec_pos_label.reshape(n_rows).astype(jnp.int32)
    ib = in_sec_pos_label.reshape(n_rows).astype(jnp.int32)
    pe2 = pe.reshape(pe.shape[0], d)
    out2 = _build_sc_call(n_rows, d)(x2, ia, ib, pe2)
    return out2.reshape(seq, batch, d)
